# Initial kernel scaffold; baseline (speedup 1.0000x reference)
import jax
import jax.numpy as jnp
from jax.experimental import pallas as pl


def kernel(adjacency, X, W, a):
    raise NotImplementedError("write your pallas kernel here")


# speedup vs baseline: 1.1478x; 1.1478x over previous
import functools

import jax
import jax.numpy as jnp
from jax.experimental import pallas as pl
from jax.experimental.pallas import tpu as pltpu

N = 10000
D = 128
ALPHA = 0.2

BM = 80


def _node_stats_kernel(x_ref, w_ref, asrc_ref, adst_ref,
                       wh_ref, fsrc_ref, u_ref, ua_ref,
                       nfdst_ref, v_ref, va_ref):
    wh = jnp.dot(x_ref[...], w_ref[...], preferred_element_type=jnp.float32)
    wh_ref[...] = wh
    fsrc = jnp.dot(wh, asrc_ref[...], preferred_element_type=jnp.float32)
    fdst = jnp.dot(wh, adst_ref[...], preferred_element_type=jnp.float32)
    fsrc_ref[...] = fsrc
    u_ref[...] = jnp.exp(-fsrc)
    ua_ref[...] = jnp.exp(-ALPHA * fsrc)
    nfdst_ref[...] = -fdst
    v_ref[...] = jnp.exp(-fdst)
    va_ref[...] = jnp.exp(-ALPHA * fdst)


def _gat_kernel(adj_ref, fsrc_ref, u_ref, ua_ref, nfd_ref, v_ref, va_ref,
                wh_ref, out_ref):
    pos = fsrc_ref[...] > nfd_ref[...]
    scale = jnp.where(pos, u_ref[...] * v_ref[...], ua_ref[...] * va_ref[...])
    e_blk = adj_ref[...] * scale

    acc = jax.lax.dot_general(
        e_blk, wh_ref[...], (((1,), (0,)), ((), ())),
        preferred_element_type=jnp.float32)
    rs = jnp.sum(e_blk, axis=1, keepdims=True)
    h = acc / rs
    out_ref[...] = jnp.where(h > 0, h, jnp.exp(h) - 1.0)


@jax.jit
def kernel(adjacency, X, W, a):
    d = W.shape[1]
    asrc = a[:, :d].T
    adst = a[:, d:].T

    bm_a = 1000
    stats_out = pl.pallas_call(
        _node_stats_kernel,
        grid=(N // bm_a,),
        in_specs=[
            pl.BlockSpec((bm_a, D), lambda i: (i, 0)),
            pl.BlockSpec((D, D), lambda i: (0, 0)),
            pl.BlockSpec((D, 1), lambda i: (0, 0)),
            pl.BlockSpec((D, 1), lambda i: (0, 0)),
        ],
        out_specs=[
            pl.BlockSpec((bm_a, D), lambda i: (i, 0)),
        ] + [pl.BlockSpec((bm_a, 1), lambda i: (i, 0))] * 6,
        out_shape=[jax.ShapeDtypeStruct((N, D), jnp.float32)]
        + [jax.ShapeDtypeStruct((N, 1), jnp.float32)] * 6,
    )(X, W, asrc, adst)
    wh, fsrc, u, ua, nfdst, v, va = stats_out

    nfd_row = nfdst.reshape(1, N)
    v_row = v.reshape(1, N)
    va_row = va.reshape(1, N)

    out = pl.pallas_call(
        _gat_kernel,
        grid=(N // BM,),
        in_specs=[
            pl.BlockSpec((BM, N), lambda i: (i, 0)),
            pl.BlockSpec((BM, 1), lambda i: (i, 0)),
            pl.BlockSpec((BM, 1), lambda i: (i, 0)),
            pl.BlockSpec((BM, 1), lambda i: (i, 0)),
            pl.BlockSpec((1, N), lambda i: (0, 0)),
            pl.BlockSpec((1, N), lambda i: (0, 0)),
            pl.BlockSpec((1, N), lambda i: (0, 0)),
            pl.BlockSpec((N, D), lambda i: (0, 0)),
        ],
        out_specs=pl.BlockSpec((BM, D), lambda i: (i, 0)),
        out_shape=jax.ShapeDtypeStruct((N, D), jnp.float32),
    )(adjacency, fsrc, u, ua, nfd_row, v_row, va_row, wh)
    return out


# speedup vs baseline: 1.1495x; 1.0015x over previous
import functools

import jax
import jax.numpy as jnp
from jax.experimental import pallas as pl
from jax.experimental.pallas import tpu as pltpu

N = 10000
D = 128
ALPHA = 0.2

BM = 80


def _node_stats_kernel(x_ref, w_ref, asrc_ref, adst_ref,
                       wh_ref, fsrc_ref, u_ref, ua_ref,
                       nfdst_ref, v_ref, va_ref):
    wh = jnp.dot(x_ref[...], w_ref[...], preferred_element_type=jnp.float32)
    wh_ref[...] = wh.astype(jnp.bfloat16)
    fsrc = jnp.dot(wh, asrc_ref[...], preferred_element_type=jnp.float32)
    fdst = jnp.dot(wh, adst_ref[...], preferred_element_type=jnp.float32)
    fsrc_ref[...] = fsrc
    u_ref[...] = jnp.exp(-fsrc)
    ua_ref[...] = jnp.exp(-ALPHA * fsrc)
    nfdst_ref[...] = -fdst
    v_ref[...] = jnp.exp(-fdst)
    va_ref[...] = jnp.exp(-ALPHA * fdst)


def _gat_kernel(adj_ref, fsrc_ref, u_ref, ua_ref, nfd_ref, v_ref, va_ref,
                wh_ref, out_ref):
    pos = fsrc_ref[...] > nfd_ref[...]
    scale = jnp.where(pos, u_ref[...] * v_ref[...], ua_ref[...] * va_ref[...])
    e_blk = adj_ref[...] * scale

    acc = jax.lax.dot_general(
        e_blk.astype(jnp.bfloat16), wh_ref[...], (((1,), (0,)), ((), ())),
        preferred_element_type=jnp.float32)
    rs = jnp.sum(e_blk, axis=1, keepdims=True)
    h = acc / rs
    out_ref[...] = jnp.where(h > 0, h, jnp.exp(h) - 1.0)


@jax.jit
def kernel(adjacency, X, W, a):
    d = W.shape[1]
    asrc = a[:, :d].T
    adst = a[:, d:].T

    bm_a = 1000
    stats_out = pl.pallas_call(
        _node_stats_kernel,
        grid=(N // bm_a,),
        in_specs=[
            pl.BlockSpec((bm_a, D), lambda i: (i, 0)),
            pl.BlockSpec((D, D), lambda i: (0, 0)),
            pl.BlockSpec((D, 1), lambda i: (0, 0)),
            pl.BlockSpec((D, 1), lambda i: (0, 0)),
        ],
        out_specs=[
            pl.BlockSpec((bm_a, D), lambda i: (i, 0)),
        ] + [pl.BlockSpec((bm_a, 1), lambda i: (i, 0))] * 6,
        out_shape=[jax.ShapeDtypeStruct((N, D), jnp.bfloat16)]
        + [jax.ShapeDtypeStruct((N, 1), jnp.float32)] * 6,
    )(X, W, asrc, adst)
    wh, fsrc, u, ua, nfdst, v, va = stats_out

    nfd_row = nfdst.reshape(1, N)
    v_row = v.reshape(1, N)
    va_row = va.reshape(1, N)

    out = pl.pallas_call(
        _gat_kernel,
        grid=(N // BM,),
        in_specs=[
            pl.BlockSpec((BM, N), lambda i: (i, 0)),
            pl.BlockSpec((BM, 1), lambda i: (i, 0)),
            pl.BlockSpec((BM, 1), lambda i: (i, 0)),
            pl.BlockSpec((BM, 1), lambda i: (i, 0)),
            pl.BlockSpec((1, N), lambda i: (0, 0)),
            pl.BlockSpec((1, N), lambda i: (0, 0)),
            pl.BlockSpec((1, N), lambda i: (0, 0)),
            pl.BlockSpec((N, D), lambda i: (0, 0)),
        ],
        out_specs=pl.BlockSpec((BM, D), lambda i: (i, 0)),
        out_shape=jax.ShapeDtypeStruct((N, D), jnp.float32),
    )(adjacency, fsrc, u, ua, nfd_row, v_row, va_row, wh)
    return out


# speedup vs baseline: 1.2398x; 1.0785x over previous
import jax
import jax.numpy as jnp
from jax.experimental import pallas as pl
from jax.experimental.pallas import tpu as pltpu

N = 10000
D = 128
DA = 256
ALPHA = 0.2

BM = 80


def _node_stats_kernel(x_ref, w_ref, asrc_ref, adst_ref,
                       whaug_ref, u_ref, ua_ref, v_ref, va_ref):
    wh = jnp.dot(x_ref[...], w_ref[...], preferred_element_type=jnp.float32)
    whaug_ref[:, :D] = wh.astype(jnp.bfloat16)
    lane = jax.lax.broadcasted_iota(jnp.int32, (x_ref.shape[0], DA - D), 1)
    whaug_ref[:, D:] = (lane == 0).astype(jnp.bfloat16)
    fsrc = jnp.dot(wh, asrc_ref[...], preferred_element_type=jnp.float32)
    fdst = jnp.dot(wh, adst_ref[...], preferred_element_type=jnp.float32)
    u_ref[...] = jnp.exp(-fsrc).astype(jnp.bfloat16)
    ua_ref[...] = jnp.exp(-ALPHA * fsrc).astype(jnp.bfloat16)
    v_ref[...] = jnp.exp(-fdst).astype(jnp.bfloat16)
    va_ref[...] = jnp.exp(-ALPHA * fdst).astype(jnp.bfloat16)


def _gat_kernel(adj_ref, u_ref, ua_ref, v_ref, va_ref, whaug_ref, out_ref):
    adj_b = adj_ref[...].astype(jnp.bfloat16)
    p1 = u_ref[...] * v_ref[...]
    p2 = ua_ref[...] * va_ref[...]
    e_b = adj_b * jnp.minimum(p1, p2)

    acc = jax.lax.dot_general(
        e_b, whaug_ref[...], (((1,), (0,)), ((), ())),
        preferred_element_type=jnp.float32)
    h = acc[:, :D] / acc[:, D:D + 1]
    out_ref[...] = jnp.where(h > 0, h, jnp.exp(h) - 1.0)


@jax.jit
def kernel(adjacency, X, W, a):
    d = W.shape[1]
    asrc = a[:, :d].T
    adst = a[:, d:].T

    bm_a = 1000
    whaug, u, ua, v, va = pl.pallas_call(
        _node_stats_kernel,
        grid=(N // bm_a,),
        in_specs=[
            pl.BlockSpec((bm_a, D), lambda i: (i, 0)),
            pl.BlockSpec((D, D), lambda i: (0, 0)),
            pl.BlockSpec((D, 1), lambda i: (0, 0)),
            pl.BlockSpec((D, 1), lambda i: (0, 0)),
        ],
        out_specs=[
            pl.BlockSpec((bm_a, DA), lambda i: (i, 0)),
        ] + [pl.BlockSpec((bm_a, 1), lambda i: (i, 0))] * 4,
        out_shape=[jax.ShapeDtypeStruct((N, DA), jnp.bfloat16)]
        + [jax.ShapeDtypeStruct((N, 1), jnp.bfloat16)] * 4,
    )(X, W, asrc, adst)

    v_row = v.reshape(1, N)
    va_row = va.reshape(1, N)

    out = pl.pallas_call(
        _gat_kernel,
        grid=(N // BM,),
        in_specs=[
            pl.BlockSpec((BM, N), lambda i: (i, 0)),
            pl.BlockSpec((BM, 1), lambda i: (i, 0)),
            pl.BlockSpec((BM, 1), lambda i: (i, 0)),
            pl.BlockSpec((1, N), lambda i: (0, 0)),
            pl.BlockSpec((1, N), lambda i: (0, 0)),
            pl.BlockSpec((N, DA), lambda i: (0, 0)),
        ],
        out_specs=pl.BlockSpec((BM, D), lambda i: (i, 0)),
        out_shape=jax.ShapeDtypeStruct((N, D), jnp.float32),
    )(adjacency, u, ua, v_row, va_row, whaug)
    return out


# speedup vs baseline: 1.8056x; 1.4564x over previous
import jax
import jax.numpy as jnp
from jax.experimental import pallas as pl
from jax.experimental.pallas import tpu as pltpu

N = 10000
D = 128
DA = 256
ALPHA = 0.2

BM = 80
K = 5


def _node_stats_kernel(x_ref, w_ref, asrc_ref, adst_ref,
                       whaug_ref, u_ref, ua_ref, v_ref, va_ref):
    wh = jnp.dot(x_ref[...], w_ref[...], preferred_element_type=jnp.float32)
    whaug_ref[:, :D] = wh.astype(jnp.bfloat16)
    lane = jax.lax.broadcasted_iota(jnp.int32, (x_ref.shape[0], DA - D), 1)
    whaug_ref[:, D:] = (lane == 0).astype(jnp.bfloat16)
    fsrc = jnp.dot(wh, asrc_ref[...], preferred_element_type=jnp.float32)
    fdst = jnp.dot(wh, adst_ref[...], preferred_element_type=jnp.float32)
    u_ref[...] = jnp.exp(-fsrc).astype(jnp.bfloat16)
    ua_ref[...] = jnp.exp(-ALPHA * fsrc).astype(jnp.bfloat16)
    v_ref[...] = jnp.exp(-fdst).astype(jnp.bfloat16)
    va_ref[...] = jnp.exp(-ALPHA * fdst).astype(jnp.bfloat16)


def _gat_kernel(*refs):
    adj_refs = refs[:K]
    u_ref, ua_ref, v_ref, va_ref, whaug_ref, out_ref = refs[K:]
    whaug = whaug_ref[...]
    for j in range(K):
        adj_b = adj_refs[j][...].astype(jnp.bfloat16)
        u = u_ref[pl.ds(j * BM, BM), :]
        ua = ua_ref[pl.ds(j * BM, BM), :]
        p1 = u * v_ref[...]
        p2 = ua * va_ref[...]
        e_b = adj_b * jnp.minimum(p1, p2)

        acc = jax.lax.dot_general(
            e_b, whaug, (((1,), (0,)), ((), ())),
            preferred_element_type=jnp.float32)
        h = acc[:, :D] / acc[:, D:D + 1]
        out_ref[pl.ds(j * BM, BM), :] = jnp.where(h > 0, h, jnp.exp(h) - 1.0)


@jax.jit
def kernel(adjacency, X, W, a):
    d = W.shape[1]
    asrc = a[:, :d].T
    adst = a[:, d:].T

    bm_a = 1000
    whaug, u, ua, v, va = pl.pallas_call(
        _node_stats_kernel,
        grid=(N // bm_a,),
        in_specs=[
            pl.BlockSpec((bm_a, D), lambda i: (i, 0)),
            pl.BlockSpec((D, D), lambda i: (0, 0)),
            pl.BlockSpec((D, 1), lambda i: (0, 0)),
            pl.BlockSpec((D, 1), lambda i: (0, 0)),
        ],
        out_specs=[
            pl.BlockSpec((bm_a, DA), lambda i: (i, 0)),
        ] + [pl.BlockSpec((bm_a, 1), lambda i: (i, 0))] * 4,
        out_shape=[jax.ShapeDtypeStruct((N, DA), jnp.bfloat16)]
        + [jax.ShapeDtypeStruct((N, 1), jnp.bfloat16)] * 4,
    )(X, W, asrc, adst)

    v_row = v.reshape(1, N)
    va_row = va.reshape(1, N)

    adj_specs = [
        pl.BlockSpec((BM, N), lambda i, j=j: (K * i + j, 0)) for j in range(K)
    ]
    out = pl.pallas_call(
        _gat_kernel,
        grid=(N // (K * BM),),
        in_specs=adj_specs + [
            pl.BlockSpec((K * BM, 1), lambda i: (i, 0)),
            pl.BlockSpec((K * BM, 1), lambda i: (i, 0)),
            pl.BlockSpec((1, N), lambda i: (0, 0)),
            pl.BlockSpec((1, N), lambda i: (0, 0)),
            pl.BlockSpec((N, DA), lambda i: (0, 0)),
        ],
        out_specs=pl.BlockSpec((K * BM, D), lambda i: (i, 0)),
        out_shape=jax.ShapeDtypeStruct((N, D), jnp.float32),
    )(*([adjacency] * K), u, ua, v_row, va_row, whaug)
    return out
